# Initial kernel scaffold; baseline (speedup 1.0000x reference)
import functools

import jax
import jax.numpy as jnp
from jax import lax
from jax.experimental import pallas as pl
from jax.experimental.pallas import tpu as pltpu
from jax.experimental.pallas import tpu_sc as plsc

_N = 10000
_E = 320000
_NC = 2
_NS = 16
_NW = _NC * _NS
_CHUNK = 128
_EPW = 10112
_NCHUNK = _EPW // _CHUNK
_EPAD = _EPW * _NW
_NACC = 10240
_RPT = _NACC // _NS

_VMESH = plsc.VectorSubcoreMesh(core_axis_name="c", subcore_axis_name="s")


def _sc_segsum(g, src, dst, zeros_w, zeros_deg, ones_rows, width, with_deg):
    out_types = [jax.ShapeDtypeStruct((_NC, _NACC, width), jnp.float32)]
    if with_deg:
        out_types.append(jax.ShapeDtypeStruct((_NC, _NACC, 16), jnp.float32))
    scratch = [
        pltpu.VMEM((_CHUNK,), jnp.int32),
        pltpu.VMEM((_CHUNK,), jnp.int32),
        pltpu.VMEM((_CHUNK, width), jnp.float32),
        pltpu.VMEM((_CHUNK, 16), jnp.float32),
        pltpu.VMEM_SHARED((_NACC, width), jnp.float32),
        pltpu.VMEM_SHARED((_NACC, 16), jnp.float32),
        pltpu.SemaphoreType.DMA,
    ]

    @functools.partial(pl.kernel, mesh=_VMESH, out_type=tuple(out_types),
                       scratch_types=scratch)
    def k(g_h, src_h, dst_h, zw_h, zd_h, ones_h, *rest):
        if with_deg:
            p_h, dp_h, idx_s, idx_d, rows_v, ones_v, acc, dacc, sem = rest
        else:
            p_h, idx_s, idx_d, rows_v, ones_v, acc, dacc, sem = rest
            dp_h = None
        c = lax.axis_index("c")
        s = lax.axis_index("s")
        wid = c * _NS + s
        r0 = s * _RPT
        pltpu.sync_copy(zw_h.at[pl.ds(r0, _RPT)], acc.at[pl.ds(r0, _RPT)])
        if with_deg:
            pltpu.sync_copy(zd_h.at[pl.ds(r0, _RPT)], dacc.at[pl.ds(r0, _RPT)])
            pltpu.sync_copy(ones_h, ones_v)
        plsc.subcore_barrier()

        base0 = wid * _EPW

        @pl.loop(0, _NCHUNK)
        def _(j):
            base = base0 + j * _CHUNK
            pltpu.sync_copy(src_h.at[pl.ds(base, _CHUNK)], idx_s)
            pltpu.sync_copy(dst_h.at[pl.ds(base, _CHUNK)], idx_d)
            pltpu.async_copy(g_h.at[idx_s], rows_v, sem).wait()
            pltpu.sync_copy(rows_v, acc.at[idx_d], add=True)
            if with_deg:
                pltpu.sync_copy(ones_v, dacc.at[idx_d], add=True)

        plsc.subcore_barrier()
        pltpu.sync_copy(acc.at[pl.ds(r0, _RPT)], p_h.at[c, pl.ds(r0, _RPT)])
        if with_deg:
            pltpu.sync_copy(dacc.at[pl.ds(r0, _RPT)], dp_h.at[c, pl.ds(r0, _RPT)])

    res = k(g, src, dst, zeros_w, zeros_deg, ones_rows)
    return res if isinstance(res, (list, tuple)) else (res,)


_BLK = 1000
_GRID = _N // _BLK


def _tc_matmul(x, w):
    dout = w.shape[1]

    def body(x_ref, w_ref, o_ref):
        o_ref[...] = jnp.dot(x_ref[...], w_ref[...],
                             preferred_element_type=jnp.float32)

    return pl.pallas_call(
        body,
        grid=(_GRID,),
        in_specs=[
            pl.BlockSpec((_BLK, x.shape[1]), lambda i: (i, 0)),
            pl.BlockSpec((x.shape[1], dout), lambda i: (0, 0)),
        ],
        out_specs=pl.BlockSpec((_BLK, dout), lambda i: (i, 0)),
        out_shape=jax.ShapeDtypeStruct((_N, dout), jnp.float32),
    )(x, w)


def _tc_combine(h, p, degp, wself, b, relu, wneigh_next=None):
    dout = wself.shape[1]
    b2 = b.reshape(1, dout)

    def body(h_ref, p0_ref, p1_ref, d0_ref, d1_ref, ws_ref, b_ref, *rest):
        if wneigh_next is None:
            (o_ref,) = rest
            wn_ref = None
        else:
            wn_ref, o_ref, g_ref = rest
        deg = d0_ref[0, :, 0:1] + d1_ref[0, :, 0:1]
        rdeg = 1.0 / jnp.maximum(deg, 1.0)
        s = (p0_ref[0] + p1_ref[0]) * rdeg
        z = jnp.dot(h_ref[...], ws_ref[...],
                    preferred_element_type=jnp.float32) + s + b_ref[...]
        if relu:
            z = jnp.maximum(z, 0.0)
        o_ref[...] = z
        if wneigh_next is not None:
            g_ref[...] = jnp.dot(z, wn_ref[...],
                                 preferred_element_type=jnp.float32)

    in_specs = [
        pl.BlockSpec((_BLK, h.shape[1]), lambda i: (i, 0)),
        pl.BlockSpec((1, _BLK, dout), lambda i: (0, i, 0)),
        pl.BlockSpec((1, _BLK, dout), lambda i: (1, i, 0)),
        pl.BlockSpec((1, _BLK, 16), lambda i: (0, i, 0)),
        pl.BlockSpec((1, _BLK, 16), lambda i: (1, i, 0)),
        pl.BlockSpec((h.shape[1], dout), lambda i: (0, 0)),
        pl.BlockSpec((1, dout), lambda i: (0, 0)),
    ]
    args = [h, p, p, degp, degp, wself, b2]
    out_shapes = [jax.ShapeDtypeStruct((_N, dout), jnp.float32)]
    out_specs = [pl.BlockSpec((_BLK, dout), lambda i: (i, 0))]
    if wneigh_next is not None:
        dnext = wneigh_next.shape[1]
        in_specs.append(pl.BlockSpec((dout, dnext), lambda i: (0, 0)))
        args.append(wneigh_next)
        out_shapes.append(jax.ShapeDtypeStruct((_N, dnext), jnp.float32))
        out_specs.append(pl.BlockSpec((_BLK, dnext), lambda i: (i, 0)))

    return pl.pallas_call(
        body,
        grid=(_GRID,),
        in_specs=in_specs,
        out_specs=out_specs,
        out_shape=out_shapes,
    )(*args)


def kernel(x, edge_index, Wself0, Wneigh0, b0, Wself1, Wneigh1, b1,
           Wself2, Wneigh2, b2):
    src = edge_index[0]
    dst = edge_index[1]
    npad = _EPAD - _E
    src_p = jnp.concatenate([src, jnp.zeros((npad,), jnp.int32)])
    dst_p = jnp.concatenate([dst, jnp.full((npad,), _N, jnp.int32)])
    z128 = jnp.zeros((_NACC, 128), jnp.float32)
    z16 = jnp.zeros((_NACC, 16), jnp.float32)
    ones_rows = jnp.ones((_CHUNK, 16), jnp.float32)

    g0 = _tc_matmul(x, Wneigh0)
    p0, degp = _sc_segsum(g0, src_p, dst_p, z128, z16, ones_rows, 128, True)
    h1, g1 = _tc_combine(x, p0, degp, Wself0, b0, True, Wneigh1)

    p1, = _sc_segsum(g1, src_p, dst_p, z128, z16, ones_rows, 128, False)
    h2, g2 = _tc_combine(h1, p1, degp, Wself1, b1, True, Wneigh2)

    p2, = _sc_segsum(g2, src_p, dst_p, z16, z16, ones_rows, 16, False)
    out = _tc_combine(h2, p2, degp, Wself2, b2, False)
    return out[0]


# speedup vs baseline: 3.5099x; 3.5099x over previous
import functools

import jax
import jax.numpy as jnp
from jax import lax
from jax.experimental import pallas as pl
from jax.experimental.pallas import tpu as pltpu
from jax.experimental.pallas import tpu_sc as plsc

_N = 10000
_E = 320000
_D = 128
_NC = 2
_NS = 16
_NW = _NC * _NS
_CHUNK = 128
_EPW = 10112
_NCHUNK = _EPW // _CHUNK
_EPAD = _EPW * _NW
_NACC = 10240
_RPT = _NACC // _NS

_VMESH = plsc.VectorSubcoreMesh(core_axis_name="c", subcore_axis_name="s")


@functools.partial(
    pl.kernel, mesh=_VMESH,
    out_type=jax.ShapeDtypeStruct((_NC, _NACC, _D), jnp.float32),
    scratch_types=[
        pltpu.VMEM((_CHUNK,), jnp.int32),
        pltpu.VMEM((_CHUNK,), jnp.int32),
        pltpu.VMEM((_CHUNK, _D), jnp.float32),
        pltpu.VMEM_SHARED((_NACC, _D), jnp.float32),
        pltpu.SemaphoreType.DMA,
    ])
def _sc_segsum(g_h, src_h, dst_h, zeros_h, p_h, idx_s, idx_d, rows_v, acc, sem):
    c = lax.axis_index("c")
    s = lax.axis_index("s")
    wid = c * _NS + s
    r0 = s * _RPT
    pltpu.sync_copy(zeros_h.at[pl.ds(r0, _RPT)], acc.at[pl.ds(r0, _RPT)])
    plsc.subcore_barrier()

    base0 = wid * _EPW

    @pl.loop(0, _NCHUNK)
    def _(j):
        base = base0 + j * _CHUNK
        pltpu.sync_copy(src_h.at[pl.ds(base, _CHUNK)], idx_s)
        pltpu.sync_copy(dst_h.at[pl.ds(base, _CHUNK)], idx_d)
        pltpu.async_copy(g_h.at[idx_s], rows_v, sem).wait()
        pltpu.sync_copy(rows_v, acc.at[idx_d], add=True)

    plsc.subcore_barrier()
    pltpu.sync_copy(acc.at[pl.ds(r0, _RPT)], p_h.at[c, pl.ds(r0, _RPT)])


@functools.partial(
    pl.kernel, mesh=_VMESH,
    out_type=jax.ShapeDtypeStruct((_NC, _NACC, _D), jnp.float32),
    scratch_types=[
        pltpu.VMEM((_CHUNK,), jnp.int32),
        pltpu.VMEM((_CHUNK, _D), jnp.float32),
        pltpu.VMEM_SHARED((_NACC, _D), jnp.float32),
        pltpu.SemaphoreType.DMA,
    ])
def _sc_degree(dst_h, zeros_h, ones_h, dp_h, idx_d, ones_v, acc, sem):
    c = lax.axis_index("c")
    s = lax.axis_index("s")
    wid = c * _NS + s
    r0 = s * _RPT
    pltpu.sync_copy(zeros_h.at[pl.ds(r0, _RPT)], acc.at[pl.ds(r0, _RPT)])
    pltpu.sync_copy(ones_h, ones_v)
    plsc.subcore_barrier()

    base0 = wid * _EPW

    @pl.loop(0, _NCHUNK)
    def _(j):
        base = base0 + j * _CHUNK
        pltpu.sync_copy(dst_h.at[pl.ds(base, _CHUNK)], idx_d)
        pltpu.sync_copy(ones_v, acc.at[idx_d], add=True)

    plsc.subcore_barrier()
    pltpu.sync_copy(acc.at[pl.ds(r0, _RPT)], dp_h.at[c, pl.ds(r0, _RPT)])


_BLK = 1000
_GRID = _N // _BLK


def _tc_matmul(x, w):
    dout = w.shape[1]

    def body(x_ref, w_ref, o_ref):
        o_ref[...] = jnp.dot(x_ref[...], w_ref[...],
                             preferred_element_type=jnp.float32)

    return pl.pallas_call(
        body,
        grid=(_GRID,),
        in_specs=[
            pl.BlockSpec((_BLK, x.shape[1]), lambda i: (i, 0)),
            pl.BlockSpec((x.shape[1], dout), lambda i: (0, 0)),
        ],
        out_specs=pl.BlockSpec((_BLK, dout), lambda i: (i, 0)),
        out_shape=jax.ShapeDtypeStruct((_N, dout), jnp.float32),
    )(x, w)


def _tc_combine(h, p, degp, wself, b, relu, wneigh_next=None, wneigh_s=None):
    dout = wself.shape[1]
    b2 = b.reshape(1, dout)

    def body(h_ref, p0_ref, p1_ref, d0_ref, d1_ref, ws_ref, b_ref, *rest):
        rest = list(rest)
        wns_ref = rest.pop(0) if wneigh_s is not None else None
        wn_ref = rest.pop(0) if wneigh_next is not None else None
        o_ref = rest.pop(0)
        g_ref = rest.pop(0) if wneigh_next is not None else None
        deg = d0_ref[0, :, 0:1] + d1_ref[0, :, 0:1]
        rdeg = 1.0 / jnp.maximum(deg, 1.0)
        mean = (p0_ref[0] + p1_ref[0]) * rdeg
        if wns_ref is not None:
            mean = jnp.dot(mean, wns_ref[...],
                           preferred_element_type=jnp.float32)
        z = jnp.dot(h_ref[...], ws_ref[...],
                    preferred_element_type=jnp.float32) + mean + b_ref[...]
        if relu:
            z = jnp.maximum(z, 0.0)
        o_ref[...] = z
        if g_ref is not None:
            g_ref[...] = jnp.dot(z, wn_ref[...],
                                 preferred_element_type=jnp.float32)

    in_specs = [
        pl.BlockSpec((_BLK, h.shape[1]), lambda i: (i, 0)),
        pl.BlockSpec((1, _BLK, _D), lambda i: (0, i, 0)),
        pl.BlockSpec((1, _BLK, _D), lambda i: (1, i, 0)),
        pl.BlockSpec((1, _BLK, _D), lambda i: (0, i, 0)),
        pl.BlockSpec((1, _BLK, _D), lambda i: (1, i, 0)),
        pl.BlockSpec((h.shape[1], dout), lambda i: (0, 0)),
        pl.BlockSpec((1, dout), lambda i: (0, 0)),
    ]
    args = [h, p, p, degp, degp, wself, b2]
    if wneigh_s is not None:
        in_specs.append(pl.BlockSpec(wneigh_s.shape, lambda i: (0, 0)))
        args.append(wneigh_s)
    out_shapes = [jax.ShapeDtypeStruct((_N, dout), jnp.float32)]
    out_specs = [pl.BlockSpec((_BLK, dout), lambda i: (i, 0))]
    if wneigh_next is not None:
        dnext = wneigh_next.shape[1]
        in_specs.append(pl.BlockSpec((dout, dnext), lambda i: (0, 0)))
        args.append(wneigh_next)
        out_shapes.append(jax.ShapeDtypeStruct((_N, dnext), jnp.float32))
        out_specs.append(pl.BlockSpec((_BLK, dnext), lambda i: (i, 0)))

    return pl.pallas_call(
        body,
        grid=(_GRID,),
        in_specs=in_specs,
        out_specs=out_specs,
        out_shape=out_shapes,
    )(*args)


def kernel(x, edge_index, Wself0, Wneigh0, b0, Wself1, Wneigh1, b1,
           Wself2, Wneigh2, b2):
    src = edge_index[0]
    dst = edge_index[1]
    npad = _EPAD - _E
    src_p = jnp.concatenate([src, jnp.zeros((npad,), jnp.int32)])
    dst_p = jnp.concatenate([dst, jnp.full((npad,), _N, jnp.int32)])
    zeros = jnp.zeros((_NACC, _D), jnp.float32)
    ones_rows = jnp.ones((_CHUNK, _D), jnp.float32)

    degp = _sc_degree(dst_p, zeros, ones_rows)

    g0 = _tc_matmul(x, Wneigh0)
    p0 = _sc_segsum(g0, src_p, dst_p, zeros)
    h1, g1 = _tc_combine(x, p0, degp, Wself0, b0, True, wneigh_next=Wneigh1)

    p1 = _sc_segsum(g1, src_p, dst_p, zeros)
    (h2,) = _tc_combine(h1, p1, degp, Wself1, b1, True)

    p2 = _sc_segsum(h2, src_p, dst_p, zeros)
    out = _tc_combine(h2, p2, degp, Wself2, b2, False, wneigh_s=Wneigh2)
    return out[0]
